# baseline (speedup 1.0000x reference)
import jax
import jax.numpy as jnp
from jax import lax
from jax.experimental import pallas as pl
from jax.experimental.pallas import tpu as pltpu
from jax.experimental.pallas import tpu_sc as plsc

DIM = 100
FEAT = 32
BATCH = 16384
NUM_CORES = 2
NUM_SUBCORES = 16
NUM_WORKERS = NUM_CORES * NUM_SUBCORES
BPW = BATCH // NUM_WORKERS
SLABS = DIM * DIM
NBUF = 16
HALF = BPW // 2
LANES = 16


def _lookup_body(i0_hbm, i1_hbm, i2_hbm, tab_hbm, out_hbm,
                 p_v, q_v, c_v, ring_v, stage_v, sems):
    wid = lax.axis_index("s") * NUM_CORES + lax.axis_index("c")
    base = pl.multiple_of(wid * BPW, BPW)
    pltpu.sync_copy(i0_hbm.at[pl.ds(base, BPW)], p_v.at[pl.ds(0, BPW)])
    pltpu.sync_copy(i1_hbm.at[pl.ds(base, BPW)], q_v)
    pltpu.sync_copy(i2_hbm.at[pl.ds(base, BPW)], c_v.at[pl.ds(0, BPW)])
    for k in range(BPW // LANES):
        s = pl.ds(k * LANES, LANES)
        p_v[s] = p_v[s] * DIM + q_v[s]

    f_lo = lax.iota(jnp.int32, LANES)
    f_hi = f_lo + LANES

    def fetch(l, slot):
        p = p_v[pl.ds(l, LANES)][0]
        pltpu.make_async_copy(
            tab_hbm.at[p], ring_v.at[slot], sems.at[slot]
        ).start()

    for h in range(2):
        off = h * HALF
        for j in range(NBUF):
            fetch(off + j, j)

        def group(g, carry):
            for j in range(NBUF):
                k = g * NBUF + j
                l = off + k
                c = c_v[pl.ds(l, LANES)][0]
                c_vec = jnp.full((LANES,), c, jnp.int32)
                pltpu.make_async_copy(
                    tab_hbm.at[0], ring_v.at[j], sems.at[j]
                ).wait()
                lo = plsc.load_gather(ring_v.at[j], [f_lo, c_vec])
                hi = plsc.load_gather(ring_v.at[j], [f_hi, c_vec])
                stage_v[k, pl.ds(0, LANES)] = lo
                stage_v[k, pl.ds(LANES, LANES)] = hi

                @pl.when(k + NBUF < HALF)
                def _():
                    fetch(l + NBUF, j)

            return carry

        lax.fori_loop(0, HALF // NBUF, group, 0)
        pltpu.sync_copy(stage_v, out_hbm.at[pl.ds(base + off, HALF)])


@jax.jit
def _lookup(i0, i1, i2, table3d):
    mesh = plsc.VectorSubcoreMesh(core_axis_name="c", subcore_axis_name="s")
    return pl.kernel(
        _lookup_body,
        out_type=jax.ShapeDtypeStruct((BATCH, FEAT), jnp.float32),
        mesh=mesh,
        compiler_params=pltpu.CompilerParams(
            use_tc_tiling_on_sc=True, needs_layout_passes=False
        ),
        scratch_types=[
            pltpu.VMEM((BPW + LANES,), jnp.int32),
            pltpu.VMEM((BPW,), jnp.int32),
            pltpu.VMEM((BPW + LANES,), jnp.int32),
            pltpu.VMEM((NBUF, FEAT, DIM), jnp.float32),
            pltpu.VMEM((HALF, FEAT), jnp.float32),
            pltpu.SemaphoreType.DMA((NBUF,)),
        ],
    )(i0, i1, i2, table3d)


def kernel(indices, table):
    idx = indices.astype(jnp.int32)
    table3d = jnp.transpose(table, (0, 1, 3, 2)).reshape(SLABS, FEAT, DIM)
    return _lookup(idx[:, 0], idx[:, 1], idx[:, 2], table3d)


# speedup vs baseline: 1.0220x; 1.0220x over previous
import jax
import jax.numpy as jnp
from jax import lax
from jax.experimental import pallas as pl
from jax.experimental.pallas import tpu as pltpu
from jax.experimental.pallas import tpu_sc as plsc

DIM = 100
FEAT = 32
BATCH = 16384
NUM_CORES = 2
NUM_SUBCORES = 16
NUM_WORKERS = NUM_CORES * NUM_SUBCORES
BPW = BATCH // NUM_WORKERS
SLABS = DIM * DIM
NBUF = 8
LANES = 16


def _lookup_body(i0_hbm, i1_hbm, i2_hbm, tab_hbm, out_hbm,
                 p_v, q_v, c_v, ring_v, stage_v, sems):
    wid = lax.axis_index("s") * NUM_CORES + lax.axis_index("c")
    base = pl.multiple_of(wid * BPW, BPW)
    pltpu.sync_copy(i0_hbm.at[pl.ds(base, BPW)], p_v.at[pl.ds(0, BPW)])
    pltpu.sync_copy(i1_hbm.at[pl.ds(base, BPW)], q_v)
    pltpu.sync_copy(i2_hbm.at[pl.ds(base, BPW)], c_v.at[pl.ds(0, BPW)])
    for k in range(BPW // LANES):
        s = pl.ds(k * LANES, LANES)
        p_v[s] = p_v[s] * DIM + q_v[s]

    f_lo = lax.iota(jnp.int32, LANES)
    f_hi = f_lo + LANES

    def fetch(l, slot):
        p = p_v[pl.ds(l, LANES)][0]
        pltpu.make_async_copy(
            tab_hbm.at[p], ring_v.at[slot], sems.at[slot]
        ).start()

    for j in range(NBUF):
        fetch(j, j)

    def group(g, carry):
        for j in range(NBUF):
            l = g * NBUF + j
            c = c_v[pl.ds(l, LANES)][0]
            c_vec = jnp.full((LANES,), c, jnp.int32)
            pltpu.make_async_copy(
                tab_hbm.at[0], ring_v.at[j], sems.at[j]
            ).wait()
            lo = plsc.load_gather(ring_v.at[j], [f_lo, c_vec])
            hi = plsc.load_gather(ring_v.at[j], [f_hi, c_vec])
            stage_v[l, pl.ds(0, LANES)] = lo
            stage_v[l, pl.ds(LANES, LANES)] = hi

            @pl.when(l + NBUF < BPW)
            def _():
                fetch(l + NBUF, j)

        return carry

    lax.fori_loop(0, BPW // NBUF, group, 0)
    pltpu.sync_copy(stage_v, out_hbm.at[pl.ds(base, BPW)])


@jax.jit
def _lookup(i0, i1, i2, table3d):
    mesh = plsc.VectorSubcoreMesh(core_axis_name="c", subcore_axis_name="s")
    return pl.kernel(
        _lookup_body,
        out_type=jax.ShapeDtypeStruct((BATCH, FEAT), jnp.float32),
        mesh=mesh,
        compiler_params=pltpu.CompilerParams(
            use_tc_tiling_on_sc=True, needs_layout_passes=False
        ),
        scratch_types=[
            pltpu.VMEM((BPW + LANES,), jnp.int32),
            pltpu.VMEM((BPW,), jnp.int32),
            pltpu.VMEM((BPW + LANES,), jnp.int32),
            pltpu.VMEM((NBUF, FEAT, DIM), jnp.float32),
            pltpu.VMEM((BPW, FEAT), jnp.float32),
            pltpu.SemaphoreType.DMA((NBUF,)),
        ],
    )(i0, i1, i2, table3d)


def kernel(indices, table):
    idx = indices.astype(jnp.int32)
    table3d = jnp.transpose(table, (0, 1, 3, 2)).reshape(SLABS, FEAT, DIM)
    return _lookup(idx[:, 0], idx[:, 1], idx[:, 2], table3d)
